# Initial kernel scaffold; baseline (speedup 1.0000x reference)
import jax
import jax.numpy as jnp
from jax.experimental import pallas as pl


def kernel(xyz, features, W1a, b1a, W1b, b1b, Wwa, bwa, Wwb, bwb):
    raise NotImplementedError("write your pallas kernel here")


# speedup vs baseline: 6.2076x; 6.2076x over previous
import functools

import jax
import jax.numpy as jnp
from jax import lax
from jax.experimental import pallas as pl
from jax.experimental.pallas import tpu as pltpu
from jax.experimental.pallas import tpu_sc as plsc

NPOINT = 512
RADIUS = 0.2
NSAMPLE = 32
ROWW = 48


def _fps_body(xc_ref, idx_ref):
    x = xc_ref[0, 0]
    y = xc_ref[1, 0]
    z = xc_ref[2, 0]
    niota = (lax.broadcasted_iota(jnp.int32, (64, 128), 0) * 128
             + lax.broadcasted_iota(jnp.int32, (64, 128), 1))

    idx_ref[0, 0:1, :] = jnp.zeros((1, 1), jnp.int32)
    px0 = x[0:1, 0:1]
    py0 = y[0:1, 0:1]
    pz0 = z[0:1, 0:1]
    dmin0 = jnp.full((64, 128), jnp.inf, jnp.float32)

    def body(i, carry):
        dmin, px, py, pz = carry
        d = (x - px) ** 2 + (y - py) ** 2 + (z - pz) ** 2
        dmin = jnp.minimum(dmin, d)
        m = jnp.max(dmin, keepdims=True)
        sel = jnp.min(jnp.where(dmin == m, niota, 1 << 20), keepdims=True)
        sel = sel.astype(jnp.int32)
        idx_ref[0, pl.ds(i, 1), :] = sel
        hit = niota == sel
        px = jnp.sum(jnp.where(hit, x, 0.0), keepdims=True)
        py = jnp.sum(jnp.where(hit, y, 0.0), keepdims=True)
        pz = jnp.sum(jnp.where(hit, z, 0.0), keepdims=True)
        return dmin, px, py, pz

    lax.fori_loop(1, NPOINT, body, (dmin0, px0, py0, pz0))


def _run_fps(xc, interpret=False):
    B = xc.shape[1]
    return pl.pallas_call(
        _fps_body,
        grid=(B,),
        in_specs=[pl.BlockSpec((3, 1, 64, 128), lambda b: (0, b, 0, 0))],
        out_specs=pl.BlockSpec((1, NPOINT, 1), lambda b: (b, 0, 0)),
        out_shape=jax.ShapeDtypeStruct((B, NPOINT, 1), jnp.int32),
        interpret=interpret,
    )(xc)


def _ballq_body(xr_ref, cent_ref, idx_ref):
    x = xr_ref[0, 0:1, :]
    y = xr_ref[0, 1:2, :]
    z = xr_ref[0, 2:3, :]
    cx = cent_ref[:, 0:1]
    cy = cent_ref[:, 1:2]
    cz = cent_ref[:, 2:3]
    dist = (cx - x) ** 2 + (cy - y) ** 2 + (cz - z) ** 2
    niota = lax.broadcasted_iota(jnp.int32, dist.shape, 1)
    kiota = lax.broadcasted_iota(jnp.int32, (dist.shape[0], NSAMPLE), 1)
    keys = jnp.where(dist <= RADIUS ** 2, dist, 1e6 + niota.astype(jnp.float32))

    def step(k, carry):
        keys, acc = carry
        m = jnp.min(keys, axis=1, keepdims=True)
        sel = jnp.min(jnp.where(keys == m, niota, 1 << 20), axis=1,
                      keepdims=True).astype(jnp.int32)
        acc = jnp.where(kiota == k, sel, acc)
        keys = jnp.where(niota == sel, jnp.inf, keys)
        return keys, acc

    _, acc = lax.fori_loop(0, NSAMPLE, step,
                           (keys, jnp.zeros((dist.shape[0], NSAMPLE), jnp.int32)))
    idx_ref[...] = acc


def _run_ballq(xr, crows, interpret=False):
    B = xr.shape[0]
    nblk = NPOINT // 128
    return pl.pallas_call(
        _ballq_body,
        grid=(B, nblk),
        in_specs=[
            pl.BlockSpec((1, 3, 8192), lambda b, j: (b, 0, 0)),
            pl.BlockSpec((128, ROWW), lambda b, j: (b * nblk + j, 0)),
        ],
        out_specs=pl.BlockSpec((128, NSAMPLE), lambda b, j: (b * nblk + j, 0)),
        out_shape=jax.ShapeDtypeStruct((B * NPOINT, NSAMPLE), jnp.int32),
        interpret=interpret,
    )(xr, crows)


def _make_sc_gather(M, nrows_table):
    info = plsc.get_sparse_core_info()
    NC, NS = info.num_cores, info.num_subcores
    NW = NC * NS
    rpw = M // NW
    chunk = min(rpw, 128)
    nchunk = rpw // chunk
    mesh = plsc.VectorSubcoreMesh(core_axis_name="c", subcore_axis_name="s")

    @functools.partial(
        pl.kernel,
        mesh=mesh,
        compiler_params=pltpu.CompilerParams(use_tc_tiling_on_sc=False),
        out_type=jax.ShapeDtypeStruct((M, ROWW), jnp.float32),
        scratch_types=[
            pltpu.VMEM((nchunk, chunk), jnp.int32),
            pltpu.VMEM((rpw, ROWW), jnp.float32),
            pltpu.SemaphoreType.DMA,
        ],
    )
    def k(table_hbm, idx_hbm, out_hbm, idx_v, rows_v, sem):
        wid = lax.axis_index("s") * NC + lax.axis_index("c")
        pltpu.sync_copy(idx_hbm.at[wid], idx_v)
        copies = [
            pltpu.async_copy(table_hbm.at[idx_v.at[j]],
                             rows_v.at[pl.ds(j * chunk, chunk)], sem)
            for j in range(nchunk)
        ]
        for c in copies:
            c.wait()
        pltpu.sync_copy(rows_v, out_hbm.at[pl.ds(wid * rpw, rpw)])

    def run(table, idx_flat):
        return k(table, idx_flat.reshape(NW, nchunk, chunk))

    return run


def _mlp_body(grows_ref, crows_ref, w1a_ref, b1a_ref, w1b_ref, b1b_ref,
              wwa3_ref, wwaf_ref, bwa_ref, wwb_ref, bwb_ref,
              nxyz_ref, fout_ref):
    BS = crows_ref.shape[0]
    R = BS * NSAMPLE
    grows = grows_ref[...]
    crows = crows_ref[...]
    colmask = lax.broadcasted_iota(jnp.int32, (BS, ROWW), 1) < 3
    cpad = jnp.where(colmask, crows, 0.0)
    cpadb = jnp.broadcast_to(cpad[:, None, :], (BS, NSAMPLE, ROWW))
    x48 = grows - cpadb.reshape(R, ROWW)

    f1 = jnp.maximum(jnp.dot(x48, w1a_ref[...],
                             preferred_element_type=jnp.float32)
                     + b1a_ref[...], 0.0)
    fp = jnp.maximum(jnp.dot(f1, w1b_ref[...],
                             preferred_element_type=jnp.float32)
                     + b1b_ref[...], 0.0)

    fp3 = fp.reshape(BS, NSAMPLE, 64)
    fm = jnp.mean(fp3, axis=1, keepdims=True)
    fc = (fp3 - fm).reshape(R, 64)

    a1 = jnp.maximum(jnp.dot(x48, wwa3_ref[...],
                             preferred_element_type=jnp.float32)
                     + jnp.dot(fc, wwaf_ref[...],
                               preferred_element_type=jnp.float32)
                     + bwa_ref[...], 0.0)
    alpha = jax.nn.sigmoid(jnp.dot(a1, wwb_ref[...],
                                   preferred_element_type=jnp.float32)
                           + bwb_ref[...])

    fout_ref[...] = jnp.sum((alpha * fp).reshape(BS, NSAMPLE, 64), axis=1)
    nxyz_ref[...] = crows[:, 0:3]


def _run_mlp(grows, crows, w1a48, b1a, w1b, b1b, wwa3_48, wwaf, bwa, wwb, bwb,
             interpret=False):
    M = crows.shape[0]
    BS = 128
    nblk = M // BS
    full = lambda i: (0, 0)
    return pl.pallas_call(
        _mlp_body,
        grid=(nblk,),
        in_specs=[
            pl.BlockSpec((BS * NSAMPLE, ROWW), lambda i: (i, 0)),
            pl.BlockSpec((BS, ROWW), lambda i: (i, 0)),
            pl.BlockSpec(w1a48.shape, full),
            pl.BlockSpec(b1a.shape, full),
            pl.BlockSpec(w1b.shape, full),
            pl.BlockSpec(b1b.shape, full),
            pl.BlockSpec(wwa3_48.shape, full),
            pl.BlockSpec(wwaf.shape, full),
            pl.BlockSpec(bwa.shape, full),
            pl.BlockSpec(wwb.shape, full),
            pl.BlockSpec(bwb.shape, full),
        ],
        out_specs=[
            pl.BlockSpec((BS, 3), lambda i: (i, 0)),
            pl.BlockSpec((BS, 64), lambda i: (i, 0)),
        ],
        out_shape=[
            jax.ShapeDtypeStruct((M, 3), jnp.float32),
            jax.ShapeDtypeStruct((M, 64), jnp.float32),
        ],
        interpret=interpret,
    )(grows, crows, w1a48, b1a, w1b, b1b, wwa3_48, wwaf, bwa, wwb, bwb)


def kernel(xyz, features, W1a, b1a, W1b, b1b, Wwa, bwa, Wwb, bwb):
    B, N, _ = xyz.shape
    D = features.shape[-1]

    xt = xyz.transpose(2, 0, 1)
    xc = xt.reshape(3, B, N // 128, 128)
    fps_idx = _run_fps(xc)[:, :, 0]

    offs = (jnp.arange(B, dtype=jnp.int32) * N)[:, None]
    fps_flat = (fps_idx + offs).reshape(B * NPOINT)

    table = jnp.concatenate(
        [xyz, features,
         jnp.zeros((B, N, ROWW - 3 - D), jnp.float32)], axis=-1
    ).reshape(B * N, ROWW)

    gather_c = _make_sc_gather(B * NPOINT, B * N)
    crows = gather_c(table, fps_flat)

    gidx = _run_ballq(xyz.transpose(0, 2, 1), crows)
    goffs = (jnp.arange(B * NPOINT, dtype=jnp.int32) // NPOINT * N)[:, None]
    gflat = (gidx + goffs).reshape(B * NPOINT * NSAMPLE)

    gather_g = _make_sc_gather(B * NPOINT * NSAMPLE, B * N)
    grows = gather_g(table, gflat)

    w1a48 = jnp.zeros((ROWW, 64), jnp.float32).at[:3 + D].set(W1a)
    wwa3_48 = jnp.zeros((ROWW, 64), jnp.float32).at[:3].set(Wwa[:3])
    wwaf = Wwa[3:]
    nxyz, fout = _run_mlp(grows, crows, w1a48, b1a[None, :], W1b,
                          b1b[None, :], wwa3_48, wwaf, bwa[None, :],
                          Wwb, bwb[None, :])
    return nxyz.reshape(B, NPOINT, 3), fout.reshape(B, NPOINT, 64)


# speedup vs baseline: 9.5588x; 1.5399x over previous
import functools

import jax
import jax.numpy as jnp
from jax import lax
from jax.experimental import pallas as pl
from jax.experimental.pallas import tpu as pltpu
from jax.experimental.pallas import tpu_sc as plsc

NPOINT = 512
RADIUS = 0.2
NSAMPLE = 32
ROWW = 48


def _fps_body(xc_ref, cxyz_ref):
    B = xc_ref.shape[1]
    x = xc_ref[0]
    y = xc_ref[1]
    z = xc_ref[2]
    sh = (B, 64, 128)
    niota = (lax.broadcasted_iota(jnp.int32, sh, 1) * 128
             + lax.broadcasted_iota(jnp.int32, sh, 2))

    px0 = x[:, 0:1, 0:1]
    py0 = y[:, 0:1, 0:1]
    pz0 = z[:, 0:1, 0:1]
    cxyz_ref[:, 0:1, 0:1] = px0
    cxyz_ref[:, 0:1, 1:2] = py0
    cxyz_ref[:, 0:1, 2:3] = pz0
    dmin0 = jnp.full(sh, jnp.inf, jnp.float32)

    def body(i, carry):
        dmin, px, py, pz = carry
        d = (x - px) ** 2 + (y - py) ** 2 + (z - pz) ** 2
        dmin = jnp.minimum(dmin, d)
        m = jnp.max(dmin, axis=(1, 2), keepdims=True)
        sel = jnp.min(jnp.where(dmin == m, niota, 1 << 20), axis=(1, 2),
                      keepdims=True)
        hit = niota == sel
        px = jnp.sum(jnp.where(hit, x, 0.0), axis=(1, 2), keepdims=True)
        py = jnp.sum(jnp.where(hit, y, 0.0), axis=(1, 2), keepdims=True)
        pz = jnp.sum(jnp.where(hit, z, 0.0), axis=(1, 2), keepdims=True)
        cxyz_ref[:, pl.ds(i, 1), 0:1] = px
        cxyz_ref[:, pl.ds(i, 1), 1:2] = py
        cxyz_ref[:, pl.ds(i, 1), 2:3] = pz
        return dmin, px, py, pz

    lax.fori_loop(1, NPOINT, body, (dmin0, px0, py0, pz0))


def _run_fps(xc, interpret=False):
    B = xc.shape[1]
    return pl.pallas_call(
        _fps_body,
        in_specs=[pl.BlockSpec(xc.shape, lambda: (0, 0, 0, 0))],
        out_specs=pl.BlockSpec((B, NPOINT, 3), lambda: (0, 0, 0)),
        out_shape=jax.ShapeDtypeStruct((B, NPOINT, 3), jnp.float32),
        interpret=interpret,
    )(xc)


def _ballq_body(xr_ref, cent_ref, idx_ref):
    x = xr_ref[0, 0:1, :]
    y = xr_ref[0, 1:2, :]
    z = xr_ref[0, 2:3, :]
    cx = cent_ref[:, 0:1]
    cy = cent_ref[:, 1:2]
    cz = cent_ref[:, 2:3]
    dist = (cx - x) ** 2 + (cy - y) ** 2 + (cz - z) ** 2
    niota = lax.broadcasted_iota(jnp.int32, dist.shape, 1)
    kiota = lax.broadcasted_iota(jnp.int32, (dist.shape[0], NSAMPLE), 1)
    keys = jnp.where(dist <= RADIUS ** 2, dist, 1e6 + niota.astype(jnp.float32))

    def step(k, carry):
        keys, acc = carry
        m = jnp.min(keys, axis=1, keepdims=True)
        sel = jnp.min(jnp.where(keys == m, niota, 1 << 20), axis=1,
                      keepdims=True).astype(jnp.int32)
        acc = jnp.where(kiota == k, sel, acc)
        keys = jnp.where(niota == sel, jnp.inf, keys)
        return keys, acc

    _, acc = lax.fori_loop(0, NSAMPLE, step,
                           (keys, jnp.zeros((dist.shape[0], NSAMPLE), jnp.int32)))
    idx_ref[...] = acc


def _run_ballq(xr, crows, interpret=False):
    B = xr.shape[0]
    nblk = NPOINT // 128
    return pl.pallas_call(
        _ballq_body,
        grid=(B, nblk),
        in_specs=[
            pl.BlockSpec((1, 3, 8192), lambda b, j: (b, 0, 0)),
            pl.BlockSpec((128, 3), lambda b, j: (b * nblk + j, 0)),
        ],
        out_specs=pl.BlockSpec((128, NSAMPLE), lambda b, j: (b * nblk + j, 0)),
        out_shape=jax.ShapeDtypeStruct((B * NPOINT, NSAMPLE), jnp.int32),
        interpret=interpret,
    )(xr, crows)


def _make_sc_gather(M, nrows_table):
    info = plsc.get_sparse_core_info()
    NC, NS = info.num_cores, info.num_subcores
    NW = NC * NS
    rpw = M // NW
    chunk = min(rpw, 128)
    nchunk = rpw // chunk
    mesh = plsc.VectorSubcoreMesh(core_axis_name="c", subcore_axis_name="s")

    @functools.partial(
        pl.kernel,
        mesh=mesh,
        compiler_params=pltpu.CompilerParams(use_tc_tiling_on_sc=False),
        out_type=jax.ShapeDtypeStruct((M, ROWW), jnp.float32),
        scratch_types=[
            pltpu.VMEM((nchunk, chunk), jnp.int32),
            pltpu.VMEM((rpw, ROWW), jnp.float32),
            pltpu.SemaphoreType.DMA,
        ],
    )
    def k(table_hbm, idx_hbm, out_hbm, idx_v, rows_v, sem):
        wid = lax.axis_index("s") * NC + lax.axis_index("c")
        pltpu.sync_copy(idx_hbm.at[wid], idx_v)
        copies = [
            pltpu.async_copy(table_hbm.at[idx_v.at[j]],
                             rows_v.at[pl.ds(j * chunk, chunk)], sem)
            for j in range(nchunk)
        ]
        for c in copies:
            c.wait()
        pltpu.sync_copy(rows_v, out_hbm.at[pl.ds(wid * rpw, rpw)])

    def run(table, idx_flat):
        return k(table, idx_flat.reshape(NW, nchunk, chunk))

    return run


def _mlp_body(grows_ref, crows_ref, w1a_ref, b1a_ref, w1b_ref, b1b_ref,
              wwa3_ref, wwaf_ref, bwa_ref, wwb_ref, bwb_ref,
              nxyz_ref, fout_ref):
    BS = crows_ref.shape[0]
    R = BS * NSAMPLE
    grows = grows_ref[...]
    crows = crows_ref[...]
    cpadb = jnp.broadcast_to(crows[:, None, :], (BS, NSAMPLE, ROWW))
    x48 = grows - cpadb.reshape(R, ROWW)

    f1 = jnp.maximum(jnp.dot(x48, w1a_ref[...],
                             preferred_element_type=jnp.float32)
                     + b1a_ref[...], 0.0)
    fp = jnp.maximum(jnp.dot(f1, w1b_ref[...],
                             preferred_element_type=jnp.float32)
                     + b1b_ref[...], 0.0)

    fp3 = fp.reshape(BS, NSAMPLE, 64)
    fm = jnp.mean(fp3, axis=1, keepdims=True)
    fc = (fp3 - fm).reshape(R, 64)

    a1 = jnp.maximum(jnp.dot(x48, wwa3_ref[...],
                             preferred_element_type=jnp.float32)
                     + jnp.dot(fc, wwaf_ref[...],
                               preferred_element_type=jnp.float32)
                     + bwa_ref[...], 0.0)
    alpha = jax.nn.sigmoid(jnp.dot(a1, wwb_ref[...],
                                   preferred_element_type=jnp.float32)
                           + bwb_ref[...])

    fout_ref[...] = jnp.sum((alpha * fp).reshape(BS, NSAMPLE, 64), axis=1)
    nxyz_ref[...] = crows[:, 0:3]


def _run_mlp(grows, crows, w1a48, b1a, w1b, b1b, wwa3_48, wwaf, bwa, wwb, bwb,
             interpret=False):
    M = crows.shape[0]
    BS = 128
    nblk = M // BS
    full = lambda i: (0, 0)
    return pl.pallas_call(
        _mlp_body,
        grid=(nblk,),
        in_specs=[
            pl.BlockSpec((BS * NSAMPLE, ROWW), lambda i: (i, 0)),
            pl.BlockSpec((BS, ROWW), lambda i: (i, 0)),
            pl.BlockSpec(w1a48.shape, full),
            pl.BlockSpec(b1a.shape, full),
            pl.BlockSpec(w1b.shape, full),
            pl.BlockSpec(b1b.shape, full),
            pl.BlockSpec(wwa3_48.shape, full),
            pl.BlockSpec(wwaf.shape, full),
            pl.BlockSpec(bwa.shape, full),
            pl.BlockSpec(wwb.shape, full),
            pl.BlockSpec(bwb.shape, full),
        ],
        out_specs=[
            pl.BlockSpec((BS, 3), lambda i: (i, 0)),
            pl.BlockSpec((BS, 64), lambda i: (i, 0)),
        ],
        out_shape=[
            jax.ShapeDtypeStruct((M, 3), jnp.float32),
            jax.ShapeDtypeStruct((M, 64), jnp.float32),
        ],
        interpret=interpret,
    )(grows, crows, w1a48, b1a, w1b, b1b, wwa3_48, wwaf, bwa, wwb, bwb)


def kernel(xyz, features, W1a, b1a, W1b, b1b, Wwa, bwa, Wwb, bwb):
    B, N, _ = xyz.shape
    D = features.shape[-1]

    xt = xyz.transpose(2, 0, 1)
    xc = xt.reshape(3, B, N // 128, 128)
    cxyz = _run_fps(xc)
    cent = cxyz.reshape(B * NPOINT, 3)

    table = jnp.concatenate(
        [xyz, features,
         jnp.zeros((B, N, ROWW - 3 - D), jnp.float32)], axis=-1
    ).reshape(B * N, ROWW)

    gidx = _run_ballq(xyz.transpose(0, 2, 1), cent)
    goffs = (jnp.arange(B * NPOINT, dtype=jnp.int32) // NPOINT * N)[:, None]
    gflat = (gidx + goffs).reshape(B * NPOINT * NSAMPLE)

    gather_g = _make_sc_gather(B * NPOINT * NSAMPLE, B * N)
    grows = gather_g(table, gflat)

    w1a48 = jnp.zeros((ROWW, 64), jnp.float32).at[:3 + D].set(W1a)
    wwa3_48 = jnp.zeros((ROWW, 64), jnp.float32).at[:3].set(Wwa[:3])
    wwaf = Wwa[3:]
    c48 = jnp.pad(cent, ((0, 0), (0, ROWW - 3)))
    nxyz, fout = _run_mlp(grows, c48, w1a48, b1a[None, :], W1b,
                          b1b[None, :], wwa3_48, wwaf, bwa[None, :],
                          Wwb, bwb[None, :])
    return nxyz.reshape(B, NPOINT, 3), fout.reshape(B, NPOINT, 64)


# speedup vs baseline: 12.9172x; 1.3514x over previous
import functools

import jax
import jax.numpy as jnp
from jax import lax
from jax.experimental import pallas as pl
from jax.experimental.pallas import tpu as pltpu
from jax.experimental.pallas import tpu_sc as plsc

NPOINT = 512
RADIUS = 0.2
NSAMPLE = 32
ROWW = 48


def _fps_body(xc_ref, cxyz_ref):
    B = xc_ref.shape[1]
    x = xc_ref[0]
    y = xc_ref[1]
    z = xc_ref[2]
    sh = (B, 64, 128)
    niota = (lax.broadcasted_iota(jnp.int32, sh, 1) * 128
             + lax.broadcasted_iota(jnp.int32, sh, 2))

    px0 = x[:, 0:1, 0:1]
    py0 = y[:, 0:1, 0:1]
    pz0 = z[:, 0:1, 0:1]
    cxyz_ref[:, 0:1, 0:1] = px0
    cxyz_ref[:, 0:1, 1:2] = py0
    cxyz_ref[:, 0:1, 2:3] = pz0
    dmin0 = jnp.full(sh, jnp.inf, jnp.float32)

    def body(i, carry):
        dmin, px, py, pz = carry
        d = (x - px) ** 2 + (y - py) ** 2 + (z - pz) ** 2
        dmin = jnp.minimum(dmin, d)
        m = jnp.max(dmin, axis=(1, 2), keepdims=True)
        sel = jnp.min(jnp.where(dmin == m, niota, 1 << 20), axis=(1, 2),
                      keepdims=True)
        hit = niota == sel
        px = jnp.sum(jnp.where(hit, x, 0.0), axis=(1, 2), keepdims=True)
        py = jnp.sum(jnp.where(hit, y, 0.0), axis=(1, 2), keepdims=True)
        pz = jnp.sum(jnp.where(hit, z, 0.0), axis=(1, 2), keepdims=True)
        cxyz_ref[:, pl.ds(i, 1), 0:1] = px
        cxyz_ref[:, pl.ds(i, 1), 1:2] = py
        cxyz_ref[:, pl.ds(i, 1), 2:3] = pz
        return dmin, px, py, pz

    lax.fori_loop(1, NPOINT, body, (dmin0, px0, py0, pz0))


def _run_fps(xc, interpret=False):
    B = xc.shape[1]
    return pl.pallas_call(
        _fps_body,
        in_specs=[pl.BlockSpec(xc.shape, lambda: (0, 0, 0, 0))],
        out_specs=pl.BlockSpec((B, NPOINT, 3), lambda: (0, 0, 0)),
        out_shape=jax.ShapeDtypeStruct((B, NPOINT, 3), jnp.float32),
        interpret=interpret,
    )(xc)


def _ballq_body(xr_ref, cent_ref, idx_ref, keys_ref):
    NC = 8192 // 128
    cx = cent_ref[:, 0:1]
    cy = cent_ref[:, 1:2]
    cz = cent_ref[:, 2:3]
    liota = lax.broadcasted_iota(jnp.int32, (128, 128), 1)
    liota_f = liota.astype(jnp.float32)
    kiota = lax.broadcasted_iota(jnp.int32, (128, NSAMPLE), 1)
    r2 = RADIUS ** 2

    cm = jnp.full((128, 128), jnp.inf, jnp.float32)
    ci = jnp.zeros((128, 128), jnp.int32)
    for c in range(NC):
        sl = pl.ds(c * 128, 128)
        xc = xr_ref[0, 0:1, sl]
        yc = xr_ref[0, 1:2, sl]
        zc = xr_ref[0, 2:3, sl]
        dc = (cx - xc) ** 2 + (cy - yc) ** 2 + (cz - zc) ** 2
        kc = jnp.where(dc <= r2, dc, (1e6 + c * 128) + liota_f)
        keys_ref[:, sl] = kc
        upd = kc < cm
        ci = jnp.where(upd, c, ci)
        cm = jnp.where(upd, kc, cm)

    def step(k, carry):
        cm, ci, acc = carry
        m = jnp.min(cm, axis=1, keepdims=True)
        seln = jnp.min(jnp.where(cm == m, ci * 128 + liota, 1 << 20),
                       axis=1, keepdims=True)
        acc = jnp.where(kiota == k, seln, acc)
        cm = jnp.full((128, 128), jnp.inf, jnp.float32)
        ci = jnp.zeros((128, 128), jnp.int32)
        for c in range(NC):
            sl = pl.ds(c * 128, 128)
            kc = keys_ref[:, sl]
            kc = jnp.where(liota == seln - c * 128, jnp.inf, kc)
            keys_ref[:, sl] = kc
            upd = kc < cm
            ci = jnp.where(upd, c, ci)
            cm = jnp.where(upd, kc, cm)
        return cm, ci, acc

    _, _, acc = lax.fori_loop(
        0, NSAMPLE, step,
        (cm, ci, jnp.zeros((128, NSAMPLE), jnp.int32)))
    idx_ref[...] = acc


def _run_ballq(xr, cent, interpret=False):
    B = xr.shape[0]
    nblk = NPOINT // 128
    return pl.pallas_call(
        _ballq_body,
        grid=(B, nblk),
        in_specs=[
            pl.BlockSpec((1, 3, 8192), lambda b, j: (b, 0, 0)),
            pl.BlockSpec((128, 3), lambda b, j: (b * nblk + j, 0)),
        ],
        out_specs=pl.BlockSpec((128, NSAMPLE), lambda b, j: (b * nblk + j, 0)),
        out_shape=jax.ShapeDtypeStruct((B * NPOINT, NSAMPLE), jnp.int32),
        scratch_shapes=[pltpu.VMEM((128, 8192), jnp.float32)],
        interpret=interpret,
    )(xr, cent)


def _make_sc_gather(M, nrows_table):
    info = plsc.get_sparse_core_info()
    NC, NS = info.num_cores, info.num_subcores
    NW = NC * NS
    rpw = M // NW
    chunk = min(rpw, 128)
    nchunk = rpw // chunk
    mesh = plsc.VectorSubcoreMesh(core_axis_name="c", subcore_axis_name="s")

    @functools.partial(
        pl.kernel,
        mesh=mesh,
        compiler_params=pltpu.CompilerParams(use_tc_tiling_on_sc=False),
        out_type=jax.ShapeDtypeStruct((M, ROWW), jnp.float32),
        scratch_types=[
            pltpu.VMEM((nchunk, chunk), jnp.int32),
            pltpu.VMEM((rpw, ROWW), jnp.float32),
            pltpu.SemaphoreType.DMA,
        ],
    )
    def k(table_hbm, idx_hbm, out_hbm, idx_v, rows_v, sem):
        wid = lax.axis_index("s") * NC + lax.axis_index("c")
        pltpu.sync_copy(idx_hbm.at[wid], idx_v)
        copies = [
            pltpu.async_copy(table_hbm.at[idx_v.at[j]],
                             rows_v.at[pl.ds(j * chunk, chunk)], sem)
            for j in range(nchunk)
        ]
        for c in copies:
            c.wait()
        pltpu.sync_copy(rows_v, out_hbm.at[pl.ds(wid * rpw, rpw)])

    def run(table, idx_flat):
        return k(table, idx_flat.reshape(NW, nchunk, chunk))

    return run


def _mlp_body(grows_ref, crows_ref, w1a_ref, b1a_ref, w1b_ref, b1b_ref,
              wwa3_ref, wwaf_ref, bwa_ref, wwb_ref, bwb_ref,
              nxyz_ref, fout_ref):
    BS = crows_ref.shape[0]
    R = BS * NSAMPLE
    grows = grows_ref[...]
    crows = crows_ref[...]
    cpadb = jnp.broadcast_to(crows[:, None, :], (BS, NSAMPLE, ROWW))
    x48 = grows - cpadb.reshape(R, ROWW)

    f1 = jnp.maximum(jnp.dot(x48, w1a_ref[...],
                             preferred_element_type=jnp.float32)
                     + b1a_ref[...], 0.0)
    fp = jnp.maximum(jnp.dot(f1, w1b_ref[...],
                             preferred_element_type=jnp.float32)
                     + b1b_ref[...], 0.0)

    fp3 = fp.reshape(BS, NSAMPLE, 64)
    fm = jnp.mean(fp3, axis=1, keepdims=True)
    fc = (fp3 - fm).reshape(R, 64)

    a1 = jnp.maximum(jnp.dot(x48, wwa3_ref[...],
                             preferred_element_type=jnp.float32)
                     + jnp.dot(fc, wwaf_ref[...],
                               preferred_element_type=jnp.float32)
                     + bwa_ref[...], 0.0)
    alpha = jax.nn.sigmoid(jnp.dot(a1, wwb_ref[...],
                                   preferred_element_type=jnp.float32)
                           + bwb_ref[...])

    fout_ref[...] = jnp.sum((alpha * fp).reshape(BS, NSAMPLE, 64), axis=1)
    nxyz_ref[...] = crows[:, 0:3]


def _run_mlp(grows, crows, w1a48, b1a, w1b, b1b, wwa3_48, wwaf, bwa, wwb, bwb,
             interpret=False):
    M = crows.shape[0]
    BS = 128
    nblk = M // BS
    full = lambda i: (0, 0)
    return pl.pallas_call(
        _mlp_body,
        grid=(nblk,),
        in_specs=[
            pl.BlockSpec((BS * NSAMPLE, ROWW), lambda i: (i, 0)),
            pl.BlockSpec((BS, ROWW), lambda i: (i, 0)),
            pl.BlockSpec(w1a48.shape, full),
            pl.BlockSpec(b1a.shape, full),
            pl.BlockSpec(w1b.shape, full),
            pl.BlockSpec(b1b.shape, full),
            pl.BlockSpec(wwa3_48.shape, full),
            pl.BlockSpec(wwaf.shape, full),
            pl.BlockSpec(bwa.shape, full),
            pl.BlockSpec(wwb.shape, full),
            pl.BlockSpec(bwb.shape, full),
        ],
        out_specs=[
            pl.BlockSpec((BS, 3), lambda i: (i, 0)),
            pl.BlockSpec((BS, 64), lambda i: (i, 0)),
        ],
        out_shape=[
            jax.ShapeDtypeStruct((M, 3), jnp.float32),
            jax.ShapeDtypeStruct((M, 64), jnp.float32),
        ],
        interpret=interpret,
    )(grows, crows, w1a48, b1a, w1b, b1b, wwa3_48, wwaf, bwa, wwb, bwb)


def kernel(xyz, features, W1a, b1a, W1b, b1b, Wwa, bwa, Wwb, bwb):
    B, N, _ = xyz.shape
    D = features.shape[-1]

    xt = xyz.transpose(2, 0, 1)
    xc = xt.reshape(3, B, N // 128, 128)
    cxyz = _run_fps(xc)
    cent = cxyz.reshape(B * NPOINT, 3)

    table = jnp.concatenate(
        [xyz, features,
         jnp.zeros((B, N, ROWW - 3 - D), jnp.float32)], axis=-1
    ).reshape(B * N, ROWW)

    gidx = _run_ballq(xyz.transpose(0, 2, 1), cent)
    goffs = (jnp.arange(B * NPOINT, dtype=jnp.int32) // NPOINT * N)[:, None]
    gflat = (gidx + goffs).reshape(B * NPOINT * NSAMPLE)

    gather_g = _make_sc_gather(B * NPOINT * NSAMPLE, B * N)
    grows = gather_g(table, gflat)

    w1a48 = jnp.zeros((ROWW, 64), jnp.float32).at[:3 + D].set(W1a)
    wwa3_48 = jnp.zeros((ROWW, 64), jnp.float32).at[:3].set(Wwa[:3])
    wwaf = Wwa[3:]
    c48 = jnp.pad(cent, ((0, 0), (0, ROWW - 3)))
    nxyz, fout = _run_mlp(grows, c48, w1a48, b1a[None, :], W1b,
                          b1b[None, :], wwa3_48, wwaf, bwa[None, :],
                          Wwb, bwb[None, :])
    return nxyz.reshape(B, NPOINT, 3), fout.reshape(B, NPOINT, 64)


# speedup vs baseline: 13.2801x; 1.0281x over previous
import functools

import jax
import jax.numpy as jnp
from jax import lax
from jax.experimental import pallas as pl
from jax.experimental.pallas import tpu as pltpu
from jax.experimental.pallas import tpu_sc as plsc

NPOINT = 512
RADIUS = 0.2
NSAMPLE = 32
ROWW = 48


def _fps_body(xc_ref, cxyz_ref):
    B = xc_ref.shape[1]
    x = xc_ref[0]
    y = xc_ref[1]
    z = xc_ref[2]
    sh = (B, 64, 128)
    niota = (lax.broadcasted_iota(jnp.int32, sh, 1) * 128
             + lax.broadcasted_iota(jnp.int32, sh, 2))

    px0 = x[:, 0:1, 0:1]
    py0 = y[:, 0:1, 0:1]
    pz0 = z[:, 0:1, 0:1]
    cxyz_ref[:, 0:1, 0:1] = px0
    cxyz_ref[:, 0:1, 1:2] = py0
    cxyz_ref[:, 0:1, 2:3] = pz0
    dmin0 = jnp.full(sh, jnp.inf, jnp.float32)

    def body(i, carry):
        dmin, px, py, pz = carry
        d = (x - px) ** 2 + (y - py) ** 2 + (z - pz) ** 2
        dmin = jnp.minimum(dmin, d)
        m = jnp.max(dmin, axis=(1, 2), keepdims=True)
        sel = jnp.min(jnp.where(dmin == m, niota, 1 << 20), axis=(1, 2),
                      keepdims=True)
        hit = niota == sel
        px = jnp.sum(jnp.where(hit, x, 0.0), axis=(1, 2), keepdims=True)
        py = jnp.sum(jnp.where(hit, y, 0.0), axis=(1, 2), keepdims=True)
        pz = jnp.sum(jnp.where(hit, z, 0.0), axis=(1, 2), keepdims=True)
        cxyz_ref[:, pl.ds(i, 1), 0:1] = px
        cxyz_ref[:, pl.ds(i, 1), 1:2] = py
        cxyz_ref[:, pl.ds(i, 1), 2:3] = pz
        return dmin, px, py, pz

    lax.fori_loop(1, NPOINT, body, (dmin0, px0, py0, pz0))


def _run_fps(xc, interpret=False):
    B = xc.shape[1]
    return pl.pallas_call(
        _fps_body,
        in_specs=[pl.BlockSpec(xc.shape, lambda: (0, 0, 0, 0))],
        out_specs=pl.BlockSpec((B, NPOINT, 3), lambda: (0, 0, 0)),
        out_shape=jax.ShapeDtypeStruct((B, NPOINT, 3), jnp.float32),
        interpret=interpret,
    )(xc)


def _ballq_body(xr_ref, cent_ref, idx_ref, keys_ref):
    NC = 8192 // 128
    RB = cent_ref.shape[0]
    cx = cent_ref[:, 0:1]
    cy = cent_ref[:, 1:2]
    cz = cent_ref[:, 2:3]
    liota = lax.broadcasted_iota(jnp.int32, (RB, 128), 1)
    liota_f = liota.astype(jnp.float32)
    kiota = lax.broadcasted_iota(jnp.int32, (RB, NSAMPLE), 1)
    r2 = RADIUS ** 2

    cm = jnp.full((RB, 128), jnp.inf, jnp.float32)
    ci = jnp.zeros((RB, 128), jnp.int32)
    for c in range(NC):
        sl = pl.ds(c * 128, 128)
        xc = xr_ref[0, 0:1, sl]
        yc = xr_ref[0, 1:2, sl]
        zc = xr_ref[0, 2:3, sl]
        dc = (cx - xc) ** 2 + (cy - yc) ** 2 + (cz - zc) ** 2
        kc = jnp.where(dc <= r2, dc, (1e6 + c * 128) + liota_f)
        keys_ref[:, sl] = kc
        upd = kc < cm
        ci = jnp.where(upd, c, ci)
        cm = jnp.where(upd, kc, cm)

    def step(k, carry):
        cm, ci, acc = carry
        m = jnp.min(cm, axis=1, keepdims=True)
        seln = jnp.min(jnp.where(cm == m, ci * 128 + liota, 1 << 20),
                       axis=1, keepdims=True)
        acc = jnp.where(kiota == k, seln, acc)
        cm = jnp.full((RB, 128), jnp.inf, jnp.float32)
        ci = jnp.zeros((RB, 128), jnp.int32)
        for c in range(NC):
            sl = pl.ds(c * 128, 128)
            kc = keys_ref[:, sl]
            kc = jnp.where(liota == seln - c * 128, jnp.inf, kc)
            keys_ref[:, sl] = kc
            upd = kc < cm
            ci = jnp.where(upd, c, ci)
            cm = jnp.where(upd, kc, cm)
        return cm, ci, acc

    _, _, acc = lax.fori_loop(
        0, NSAMPLE, step,
        (cm, ci, jnp.zeros((RB, NSAMPLE), jnp.int32)))
    idx_ref[...] = acc


def _run_ballq(xr, cent, interpret=False):
    B = xr.shape[0]
    RB = 64
    nblk = NPOINT // RB
    return pl.pallas_call(
        _ballq_body,
        grid=(B, nblk),
        in_specs=[
            pl.BlockSpec((1, 3, 8192), lambda b, j: (b, 0, 0)),
            pl.BlockSpec((RB, 3), lambda b, j: (b * nblk + j, 0)),
        ],
        out_specs=pl.BlockSpec((RB, NSAMPLE), lambda b, j: (b * nblk + j, 0)),
        out_shape=jax.ShapeDtypeStruct((B * NPOINT, NSAMPLE), jnp.int32),
        scratch_shapes=[pltpu.VMEM((RB, 8192), jnp.float32)],
        interpret=interpret,
    )(xr, cent)


def _make_sc_gather(M, nrows_table):
    info = plsc.get_sparse_core_info()
    NC, NS = info.num_cores, info.num_subcores
    NW = NC * NS
    rpw = M // NW
    chunk = min(rpw, 128)
    nchunk = rpw // chunk
    mesh = plsc.VectorSubcoreMesh(core_axis_name="c", subcore_axis_name="s")

    @functools.partial(
        pl.kernel,
        mesh=mesh,
        compiler_params=pltpu.CompilerParams(use_tc_tiling_on_sc=False),
        out_type=jax.ShapeDtypeStruct((M, ROWW), jnp.float32),
        scratch_types=[
            pltpu.VMEM((nchunk, chunk), jnp.int32),
            pltpu.VMEM((rpw, ROWW), jnp.float32),
            pltpu.SemaphoreType.DMA,
        ],
    )
    def k(table_hbm, idx_hbm, out_hbm, idx_v, rows_v, sem):
        wid = lax.axis_index("s") * NC + lax.axis_index("c")
        pltpu.sync_copy(idx_hbm.at[wid], idx_v)
        copies = [
            pltpu.async_copy(table_hbm.at[idx_v.at[j]],
                             rows_v.at[pl.ds(j * chunk, chunk)], sem)
            for j in range(nchunk)
        ]
        for c in copies:
            c.wait()
        pltpu.sync_copy(rows_v, out_hbm.at[pl.ds(wid * rpw, rpw)])

    def run(table, idx_flat):
        return k(table, idx_flat.reshape(NW, nchunk, chunk))

    return run


def _mlp_body(grows_ref, crows_ref, w1a_ref, b1a_ref, w1b_ref, b1b_ref,
              wwa3_ref, wwaf_ref, bwa_ref, wwb_ref, bwb_ref,
              nxyz_ref, fout_ref):
    BS = crows_ref.shape[0]
    R = BS * NSAMPLE
    grows = grows_ref[...]
    crows = crows_ref[...]
    cpadb = jnp.broadcast_to(crows[:, None, :], (BS, NSAMPLE, ROWW))
    x48 = grows - cpadb.reshape(R, ROWW)

    f1 = jnp.maximum(jnp.dot(x48, w1a_ref[...],
                             preferred_element_type=jnp.float32)
                     + b1a_ref[...], 0.0)
    fp = jnp.maximum(jnp.dot(f1, w1b_ref[...],
                             preferred_element_type=jnp.float32)
                     + b1b_ref[...], 0.0)

    fp3 = fp.reshape(BS, NSAMPLE, 64)
    fm = jnp.mean(fp3, axis=1, keepdims=True)
    fc = (fp3 - fm).reshape(R, 64)

    a1 = jnp.maximum(jnp.dot(x48, wwa3_ref[...],
                             preferred_element_type=jnp.float32)
                     + jnp.dot(fc, wwaf_ref[...],
                               preferred_element_type=jnp.float32)
                     + bwa_ref[...], 0.0)
    alpha = jax.nn.sigmoid(jnp.dot(a1, wwb_ref[...],
                                   preferred_element_type=jnp.float32)
                           + bwb_ref[...])

    fout_ref[...] = jnp.sum((alpha * fp).reshape(BS, NSAMPLE, 64), axis=1)
    nxyz_ref[...] = crows[:, 0:3]


def _run_mlp(grows, crows, w1a48, b1a, w1b, b1b, wwa3_48, wwaf, bwa, wwb, bwb,
             interpret=False):
    M = crows.shape[0]
    BS = 128
    nblk = M // BS
    full = lambda i: (0, 0)
    return pl.pallas_call(
        _mlp_body,
        grid=(nblk,),
        in_specs=[
            pl.BlockSpec((BS * NSAMPLE, ROWW), lambda i: (i, 0)),
            pl.BlockSpec((BS, ROWW), lambda i: (i, 0)),
            pl.BlockSpec(w1a48.shape, full),
            pl.BlockSpec(b1a.shape, full),
            pl.BlockSpec(w1b.shape, full),
            pl.BlockSpec(b1b.shape, full),
            pl.BlockSpec(wwa3_48.shape, full),
            pl.BlockSpec(wwaf.shape, full),
            pl.BlockSpec(bwa.shape, full),
            pl.BlockSpec(wwb.shape, full),
            pl.BlockSpec(bwb.shape, full),
        ],
        out_specs=[
            pl.BlockSpec((BS, 3), lambda i: (i, 0)),
            pl.BlockSpec((BS, 64), lambda i: (i, 0)),
        ],
        out_shape=[
            jax.ShapeDtypeStruct((M, 3), jnp.float32),
            jax.ShapeDtypeStruct((M, 64), jnp.float32),
        ],
        interpret=interpret,
    )(grows, crows, w1a48, b1a, w1b, b1b, wwa3_48, wwaf, bwa, wwb, bwb)


def kernel(xyz, features, W1a, b1a, W1b, b1b, Wwa, bwa, Wwb, bwb):
    B, N, _ = xyz.shape
    D = features.shape[-1]

    xt = xyz.transpose(2, 0, 1)
    xc = xt.reshape(3, B, N // 128, 128)
    cxyz = _run_fps(xc)
    cent = cxyz.reshape(B * NPOINT, 3)

    table = jnp.concatenate(
        [xyz, features,
         jnp.zeros((B, N, ROWW - 3 - D), jnp.float32)], axis=-1
    ).reshape(B * N, ROWW)

    gidx = _run_ballq(xyz.transpose(0, 2, 1), cent)
    goffs = (jnp.arange(B * NPOINT, dtype=jnp.int32) // NPOINT * N)[:, None]
    gflat = (gidx + goffs).reshape(B * NPOINT * NSAMPLE)

    gather_g = _make_sc_gather(B * NPOINT * NSAMPLE, B * N)
    grows = gather_g(table, gflat)

    w1a48 = jnp.zeros((ROWW, 64), jnp.float32).at[:3 + D].set(W1a)
    wwa3_48 = jnp.zeros((ROWW, 64), jnp.float32).at[:3].set(Wwa[:3])
    wwaf = Wwa[3:]
    c48 = jnp.pad(cent, ((0, 0), (0, ROWW - 3)))
    nxyz, fout = _run_mlp(grows, c48, w1a48, b1a[None, :], W1b,
                          b1b[None, :], wwa3_48, wwaf, bwa[None, :],
                          Wwb, bwb[None, :])
    return nxyz.reshape(B, NPOINT, 3), fout.reshape(B, NPOINT, 64)
